# baseline (speedup 1.0000x reference)
import functools

import jax
import jax.numpy as jnp
import numpy as np
from jax import lax
from jax.experimental import pallas as pl
from jax.experimental.pallas import tpu as pltpu
from jax.experimental.pallas import tpu_sc as plsc

NC = 2
NS = 16
LN = 16
NW = NC * NS

G = 80


def _silu(x):
    return x * jax.nn.sigmoid(x)


def _pq(h, We1, be1):
    N, D = h.shape
    H = We1.shape[1]
    BN = 2000

    def body(h_ref, wa_ref, wb_ref, b_ref, p_ref, q_ref):
        hh = h_ref[...]
        p_ref[...] = jnp.dot(hh, wa_ref[...], preferred_element_type=jnp.float32) + b_ref[...]
        q_ref[...] = jnp.dot(hh, wb_ref[...], preferred_element_type=jnp.float32)

    return pl.pallas_call(
        body,
        grid=(N // BN,),
        in_specs=[
            pl.BlockSpec((BN, D), lambda i: (i, 0)),
            pl.BlockSpec((D, H), lambda i: (0, 0)),
            pl.BlockSpec((D, H), lambda i: (0, 0)),
            pl.BlockSpec((1, H), lambda i: (0, 0)),
        ],
        out_specs=[
            pl.BlockSpec((BN, H), lambda i: (i, 0)),
            pl.BlockSpec((BN, H), lambda i: (i, 0)),
        ],
        out_shape=[
            jax.ShapeDtypeStruct((N, H), jnp.float32),
            jax.ShapeDtypeStruct((N, H), jnp.float32),
        ],
    )(h, We1[:D], We1[D:2 * D], be1.reshape(1, H))


def _sc_gather(P, Q, coordT, row, col):
    N, H = P.shape
    E = row.shape[0]
    EPW = E // NW
    G2 = 40
    nch = EPW // G2
    NPAIR = nch // 2
    mesh = plsc.VectorSubcoreMesh(core_axis_name="c", subcore_axis_name="s")

    H2 = H // 2
    vec_e = jax.ShapeDtypeStruct((E,), jnp.float32)
    buf_gh = pltpu.VMEM((G2, H), jnp.float32)
    buf_ob = pltpu.VMEM((G2, H2), jnp.int32)
    buf48 = pltpu.VMEM((48,), jnp.float32)

    @functools.partial(
        pl.kernel,
        out_type=[jax.ShapeDtypeStruct((E, H2), jnp.int32), vec_e, vec_e, vec_e, vec_e],
        mesh=mesh,
        compiler_params=pltpu.CompilerParams(needs_layout_passes=False),
        scratch_types=[
            pltpu.VMEM((EPW + 16,), jnp.int32),
            pltpu.VMEM((EPW + 16,), jnp.int32),
            pltpu.VMEM((3 * N,), jnp.float32),
            buf_gh, buf_gh, buf_gh, buf_gh, buf_ob, buf_ob,
            buf48, buf48, buf48, buf48, buf48, buf48, buf48, buf48,
            pltpu.SemaphoreType.DMA, pltpu.SemaphoreType.DMA,
            pltpu.SemaphoreType.DMA, pltpu.SemaphoreType.DMA,
        ],
    )
    def k(p_hbm, q_hbm, ct_hbm, row_hbm, col_hbm,
          pre_hbm, rad_hbm, dx_hbm, dy_hbm, dz_hbm,
          idxr, idxc, cv,
          pb0, pb1, qb0, qb1, ob0, ob1,
          rb0, rb1, xb0, xb1, yb0, yb1, zb0, zb1,
          semg0, semg1, semw0, semw1):
        wid = lax.axis_index("c") * NS + lax.axis_index("s")
        base0 = wid * EPW
        n1 = jnp.full((LN,), N, jnp.int32)
        n2 = jnp.full((LN,), 2 * N, jnp.int32)

        pbs, qbs, obs = (pb0, pb1), (qb0, qb1), (ob0, ob1)
        rbs, xbs, ybs, zbs = (rb0, rb1), (xb0, xb1), (yb0, yb1), (zb0, zb1)
        semgs, semws = (semg0, semg1), (semw0, semw1)

        pltpu.sync_copy(row_hbm.at[pl.ds(base0, EPW)], idxr.at[pl.ds(0, EPW)])
        pltpu.sync_copy(col_hbm.at[pl.ds(base0, EPW)], idxc.at[pl.ds(0, EPW)])
        idxr[pl.ds(EPW, LN)] = jnp.zeros((LN,), jnp.int32)
        idxc[pl.ds(EPW, LN)] = jnp.zeros((LN,), jnp.int32)
        pltpu.sync_copy(ct_hbm, cv)

        def issue_gather(kd, b):
            pltpu.async_copy(p_hbm.at[idxr.at[pl.ds(kd * G2, G2)]], pbs[b], semgs[b])
            pltpu.async_copy(q_hbm.at[idxc.at[pl.ds(kd * G2, G2)]], qbs[b], semgs[b])

        def drain_gather(b):
            pltpu.make_async_copy(p_hbm.at[idxr.at[pl.ds(0, G2)]], pbs[b], semgs[b]).wait()
            pltpu.make_async_copy(q_hbm.at[idxc.at[pl.ds(0, G2)]], qbs[b], semgs[b]).wait()

        def drain_write(b):
            pltpu.make_async_copy(obs[b], pre_hbm.at[pl.ds(base0, G2)], semws[b]).wait()
            pltpu.make_async_copy(rbs[b].at[pl.ds(0, G2)], rad_hbm.at[pl.ds(base0, G2)], semws[b]).wait()
            pltpu.make_async_copy(xbs[b].at[pl.ds(0, G2)], dx_hbm.at[pl.ds(base0, G2)], semws[b]).wait()
            pltpu.make_async_copy(ybs[b].at[pl.ds(0, G2)], dy_hbm.at[pl.ds(base0, G2)], semws[b]).wait()
            pltpu.make_async_copy(zbs[b].at[pl.ds(0, G2)], dz_hbm.at[pl.ds(base0, G2)], semws[b]).wait()

        def stage(kd, b, first, issue_next):
            if not first:
                drain_write(b)
            rb, xb, yb, zb = rbs[b], xbs[b], ybs[b], zbs[b]
            base_l = kd * G2
            for g in range(3):
                lsl = pl.ds(base_l + g * LN, LN)
                r16 = idxr[lsl]
                c16 = idxc[lsl]
                dx = plsc.load_gather(cv, [r16]) - plsc.load_gather(cv, [c16])
                dy = plsc.load_gather(cv, [r16 + n1]) - plsc.load_gather(cv, [c16 + n1])
                dz = plsc.load_gather(cv, [r16 + n2]) - plsc.load_gather(cv, [c16 + n2])
                osl = pl.ds(g * LN, LN)
                xb[osl] = dx
                yb[osl] = dy
                zb[osl] = dz
                rb[osl] = dx * dx + dy * dy + dz * dz
            drain_gather(b)
            pb, qb, ob = pbs[b], qbs[b], obs[b]

            def addrow(r, c2):
                for j in range(H // (2 * LN)):
                    slo = pl.ds(2 * j * LN, LN)
                    shi = pl.ds((2 * j + 1) * LN, LN)
                    lo = pb[r, slo] + qb[r, slo]
                    hi = pb[r, shi] + qb[r, shi]
                    packed = plsc.pack(lo, hi, format=plsc.PackFormat.INTERLEAVED)
                    ob[r, pl.ds(j * LN, LN)] = plsc.bitcast(packed, jnp.int32)
                return c2

            lax.fori_loop(0, G2, addrow, 0)
            if issue_next:
                issue_gather(kd + 2, b)
            base = base0 + kd * G2
            pltpu.async_copy(ob, pre_hbm.at[pl.ds(base, G2)], semws[b])
            pltpu.async_copy(rb.at[pl.ds(0, G2)], rad_hbm.at[pl.ds(base, G2)], semws[b])
            pltpu.async_copy(xb.at[pl.ds(0, G2)], dx_hbm.at[pl.ds(base, G2)], semws[b])
            pltpu.async_copy(yb.at[pl.ds(0, G2)], dy_hbm.at[pl.ds(base, G2)], semws[b])
            pltpu.async_copy(zb.at[pl.ds(0, G2)], dz_hbm.at[pl.ds(base, G2)], semws[b])

        issue_gather(0, 0)
        issue_gather(1, 1)
        stage(0, 0, True, True)
        stage(1, 1, True, True)

        def pair(kk, c2):
            stage(2 * kk, 0, False, True)
            stage(2 * kk + 1, 1, False, True)
            return c2

        lax.fori_loop(1, NPAIR - 1, pair, 0)
        stage(nch - 2, 0, False, False)
        stage(nch - 1, 1, False, False)
        drain_write(0)
        drain_write(1)

    return k(P, Q, coordT, row, col)


def _edge_mlp(pre, rad, mask, wr, We2, be2, Wc1, bc1, Wc2):
    E, H = pre.shape
    BE = 2560

    def body(pre_ref, rad_ref, m_ref, wr_ref, w2_ref, b2_ref,
             wc1_ref, bc1_ref, wc2_ref, ef_ref, s_ref):
        pre1 = pre_ref[...].astype(jnp.float32) + rad_ref[...] * wr_ref[...]
        a1 = _silu(pre1)
        a2 = _silu(jnp.dot(a1, w2_ref[...], preferred_element_type=jnp.float32) + b2_ref[...])
        ef = a2 * m_ref[...]
        c1 = _silu(jnp.dot(ef, wc1_ref[...], preferred_element_type=jnp.float32) + bc1_ref[...])
        sv = jnp.dot(c1, wc2_ref[...], preferred_element_type=jnp.float32) * m_ref[...]
        ef_ref[...] = ef
        s_ref[...] = sv

    full = lambda i: (0, 0)
    return pl.pallas_call(
        body,
        grid=(E // BE,),
        in_specs=[
            pl.BlockSpec((BE, H), lambda i: (i, 0)),
            pl.BlockSpec((BE, 1), lambda i: (i, 0)),
            pl.BlockSpec((BE, 1), lambda i: (i, 0)),
            pl.BlockSpec((1, H), full),
            pl.BlockSpec((H, H), full),
            pl.BlockSpec((1, H), full),
            pl.BlockSpec((H, H), full),
            pl.BlockSpec((1, H), full),
            pl.BlockSpec((H, 1), full),
        ],
        out_specs=[
            pl.BlockSpec((BE, H), lambda i: (i, 0)),
            pl.BlockSpec((BE, 1), lambda i: (i, 0)),
        ],
        out_shape=[
            jax.ShapeDtypeStruct((E, H), jnp.float32),
            jax.ShapeDtypeStruct((E, 1), jnp.float32),
        ],
    )(pre, rad.reshape(E, 1), mask, wr, We2, be2.reshape(1, H), Wc1, bc1.reshape(1, H), Wc2)


def _sc_scatter(ef, s, dx, dy, dz, row, N):
    E, H = ef.shape
    EPW = E // NW
    G2 = 40
    nch = EPW // G2
    NP = ((N + 40 * NS - 1) // (40 * NS)) * (40 * NS)
    RPT = NP // NS
    ZR = 40
    XW = 8
    ZC = XW * RPT // 4
    mesh = plsc.VectorSubcoreMesh(core_axis_name="c", subcore_axis_name="s")

    ibuf40 = pltpu.VMEM((G2,), jnp.int32)
    ibuf48 = pltpu.VMEM((48,), jnp.int32)
    fbuf48 = pltpu.VMEM((48,), jnp.float32)

    @functools.partial(
        pl.kernel,
        out_type=[
            jax.ShapeDtypeStruct((NC, NP, H), jnp.float32),
            jax.ShapeDtypeStruct((NC, XW * NP), jnp.float32),
        ],
        mesh=mesh,
        compiler_params=pltpu.CompilerParams(needs_layout_passes=False),
        scratch_types=[
            ibuf48, ibuf48,
            fbuf48, fbuf48, fbuf48, fbuf48, fbuf48, fbuf48, fbuf48, fbuf48,
            pltpu.VMEM((G2, H), jnp.float32),
            pltpu.VMEM((G2, H), jnp.float32),
            ibuf40, ibuf40, ibuf40, ibuf40, ibuf40, ibuf40, ibuf40, ibuf40,
            fbuf48, fbuf48, fbuf48, fbuf48, fbuf48, fbuf48, fbuf48,
            pltpu.VMEM((ZR, H), jnp.float32),
            pltpu.VMEM((ZC,), jnp.float32),
            pltpu.VMEM_SHARED((NP, H), jnp.float32),
            pltpu.VMEM_SHARED((XW * NP,), jnp.float32),
            pltpu.SemaphoreType.DMA, pltpu.SemaphoreType.DMA,
            pltpu.SemaphoreType.DMA, pltpu.SemaphoreType.DMA,
        ],
    )
    def k(ef_hbm, s_hbm, dx_hbm, dy_hbm, dz_hbm, row_hbm,
          aggh_hbm, aggc_hbm,
          rw0, rw1, sb0, sb1, xb0, xb1, yb0, yb1, zx0, zx1,
          ef0, ef1,
          ix0, ix1, iy0, iy1, iz0, iz1, ic0, ic1,
          px0, px1, py0, py1, pz0, pz1, pone,
          zb, zcb, acc_h, acc_c,
          semr0, semr1, sems0, sems1):
        cid = lax.axis_index("c")
        sid = lax.axis_index("s")
        wid = cid * NS + sid
        base0 = wid * EPW
        fz16 = jnp.zeros((LN,), jnp.float32)
        ones16 = jnp.ones((LN,), jnp.float32)
        np1 = jnp.full((LN,), NP, jnp.int32)
        np2 = jnp.full((LN,), 2 * NP, jnp.int32)
        np3 = jnp.full((LN,), 3 * NP, jnp.int32)
        rws, sbs = (rw0, rw1), (sb0, sb1)
        xbs, ybs, zxs = (xb0, xb1), (yb0, yb1), (zx0, zx1)
        efs = (ef0, ef1)
        ixs, iys, izs, ics = (ix0, ix1), (iy0, iy1), (iz0, iz1), (ic0, ic1)
        pxs, pys, pzs = (px0, px1), (py0, py1), (pz0, pz1)
        semrs, semss = (semr0, semr1), (sems0, sems1)

        def zrow(r, c2):
            for j in range(H // LN):
                zb[r, pl.ds(j * LN, LN)] = fz16
            return c2

        lax.fori_loop(0, ZR, zrow, 0)

        def zflat(i, c2):
            zcb[pl.ds(i * LN, LN)] = fz16
            return c2

        lax.fori_loop(0, ZC // LN, zflat, 0)
        for g in range(3):
            pone[pl.ds(g * LN, LN)] = ones16

        r0 = sid * RPT
        for t in range(RPT // ZR):
            pltpu.sync_copy(zb, acc_h.at[pl.ds(r0 + t * ZR, ZR)])
        for t in range(4):
            pltpu.sync_copy(zcb, acc_c.at[pl.ds(sid * XW * RPT + t * ZC, ZC)])
        plsc.subcore_barrier()

        iota16 = lax.iota(jnp.int32, LN)
        tailm = iota16 < (G2 - 2 * LN)
        tidx = iota16 + 2 * LN

        def issue_reads(kd, b):
            base = base0 + kd * G2
            pltpu.async_copy(ef_hbm.at[pl.ds(base, G2)], efs[b], semrs[b])
            pltpu.async_copy(row_hbm.at[pl.ds(base, G2)], rws[b].at[pl.ds(0, G2)], semrs[b])
            pltpu.async_copy(s_hbm.at[pl.ds(base, G2)], sbs[b].at[pl.ds(0, G2)], semrs[b])
            pltpu.async_copy(dx_hbm.at[pl.ds(base, G2)], xbs[b].at[pl.ds(0, G2)], semrs[b])
            pltpu.async_copy(dy_hbm.at[pl.ds(base, G2)], ybs[b].at[pl.ds(0, G2)], semrs[b])
            pltpu.async_copy(dz_hbm.at[pl.ds(base, G2)], zxs[b].at[pl.ds(0, G2)], semrs[b])

        def drain_reads(b):
            pltpu.make_async_copy(ef_hbm.at[pl.ds(base0, G2)], efs[b], semrs[b]).wait()
            pltpu.make_async_copy(row_hbm.at[pl.ds(base0, G2)], rws[b].at[pl.ds(0, G2)], semrs[b]).wait()
            pltpu.make_async_copy(s_hbm.at[pl.ds(base0, G2)], sbs[b].at[pl.ds(0, G2)], semrs[b]).wait()
            pltpu.make_async_copy(dx_hbm.at[pl.ds(base0, G2)], xbs[b].at[pl.ds(0, G2)], semrs[b]).wait()
            pltpu.make_async_copy(dy_hbm.at[pl.ds(base0, G2)], ybs[b].at[pl.ds(0, G2)], semrs[b]).wait()
            pltpu.make_async_copy(dz_hbm.at[pl.ds(base0, G2)], zxs[b].at[pl.ds(0, G2)], semrs[b]).wait()

        def drain_adds(b):
            pltpu.make_async_copy(efs[b], acc_h.at[ixs[b]], semss[b]).wait()
            pltpu.make_async_copy(pxs[b].at[pl.ds(0, G2)], acc_c.at[ixs[b]], semss[b]).wait()
            pltpu.make_async_copy(pys[b].at[pl.ds(0, G2)], acc_c.at[iys[b]], semss[b]).wait()
            pltpu.make_async_copy(pzs[b].at[pl.ds(0, G2)], acc_c.at[izs[b]], semss[b]).wait()
            pltpu.make_async_copy(pone.at[pl.ds(0, G2)], acc_c.at[ics[b]], semss[b]).wait()

        def stage(kd, b, first, issue_next):
            if not first:
                drain_adds(1 - b)
            if issue_next:
                issue_reads(kd + 1, 1 - b)
            ixv, iyv, izv, icv = ixs[b], iys[b], izs[b], ics[b]
            pxv, pyv, pzv = pxs[b], pys[b], pzs[b]
            drain_reads(b)
            for g in range(3):
                lsl = pl.ds(g * LN, LN)
                v16 = rws[b][lsl]
                sv = sbs[b][lsl]
                pxe = xbs[b][lsl] * sv
                pye = ybs[b][lsl] * sv
                pze = zxs[b][lsl] * sv
                if g < 2:
                    ixv[lsl] = v16
                    iyv[lsl] = v16 + np1
                    izv[lsl] = v16 + np2
                    icv[lsl] = v16 + np3
                    pxv[lsl] = pxe
                    pyv[lsl] = pye
                    pzv[lsl] = pze
                else:
                    plsc.store_scatter(ixv, [tidx], v16, mask=tailm)
                    plsc.store_scatter(iyv, [tidx], v16 + np1, mask=tailm)
                    plsc.store_scatter(izv, [tidx], v16 + np2, mask=tailm)
                    plsc.store_scatter(icv, [tidx], v16 + np3, mask=tailm)
                    pxv[lsl] = pxe
                    pyv[lsl] = pye
                    pzv[lsl] = pze
            pltpu.async_copy(efs[b], acc_h.at[ixv], semss[b], add=True)
            pltpu.async_copy(pxv.at[pl.ds(0, G2)], acc_c.at[ixv], semss[b], add=True)
            pltpu.async_copy(pyv.at[pl.ds(0, G2)], acc_c.at[iyv], semss[b], add=True)
            pltpu.async_copy(pzv.at[pl.ds(0, G2)], acc_c.at[izv], semss[b], add=True)
            pltpu.async_copy(pone.at[pl.ds(0, G2)], acc_c.at[icv], semss[b], add=True)

        issue_reads(0, 0)
        stage(0, 0, True, True)

        def pair(kk, c2):
            stage(2 * kk + 1, 1, False, True)
            stage(2 * kk + 2, 0, False, True)
            return c2

        lax.fori_loop(0, (nch - 2) // 2, pair, 0)
        stage(nch - 1, 1, False, False)
        drain_adds(1)
        plsc.subcore_barrier()
        pltpu.sync_copy(acc_h.at[pl.ds(r0, RPT)], aggh_hbm.at[cid, pl.ds(r0, RPT)])
        pltpu.sync_copy(acc_c.at[pl.ds(sid * XW * RPT, XW * RPT)],
                        aggc_hbm.at[cid, pl.ds(sid * XW * RPT, XW * RPT)])

    return k(ef, s, dx, dy, dz, row)


def _node_mlp(h, parts, coordT4, Wn1, bn1, Wn2, bn2):
    NP, D = h.shape
    H = Wn2.shape[0]
    XW = parts[0][1].shape[1]
    NPART = len(parts)
    BN = 2048

    def body(*refs):
        h_ref = refs[0]
        part_refs = refs[1:1 + 2 * NPART]
        cp_ref, wa_ref, wb_ref, b1_ref, w2_ref, b2_ref, ho_ref, co_ref = refs[1 + 2 * NPART:]
        agg = 0.0
        c4 = 0.0
        for i in range(NPART):
            agg = agg + part_refs[2 * i][0] + part_refs[2 * i][1]
            c4 = c4 + part_refs[2 * i + 1][0] + part_refs[2 * i + 1][1]
        x = _silu(jnp.dot(h_ref[...], wa_ref[...], preferred_element_type=jnp.float32)
                  + jnp.dot(agg, wb_ref[...], preferred_element_type=jnp.float32)
                  + b1_ref[...])
        ho_ref[...] = jnp.dot(x, w2_ref[...], preferred_element_type=jnp.float32) + b2_ref[...]
        cnt = jnp.maximum(c4[3:4, :], 1.0)
        co_ref[...] = cp_ref[...] + c4 / cnt

    full = lambda i: (0, 0)
    part_specs = []
    part_args = []
    for aggh, aggc in parts:
        part_specs.append(pl.BlockSpec((2, BN, H), lambda i: (0, i, 0)))
        part_specs.append(pl.BlockSpec((2, XW, BN), lambda i: (0, 0, i)))
        part_args.extend([aggh, aggc])
    return pl.pallas_call(
        body,
        grid=(NP // BN,),
        in_specs=[pl.BlockSpec((BN, D), lambda i: (i, 0))] + part_specs + [
            pl.BlockSpec((XW, BN), lambda i: (0, i)),
            pl.BlockSpec((D, H), full),
            pl.BlockSpec((H, H), full),
            pl.BlockSpec((1, H), full),
            pl.BlockSpec((H, D), full),
            pl.BlockSpec((1, D), full),
        ],
        out_specs=[
            pl.BlockSpec((BN, D), lambda i: (i, 0)),
            pl.BlockSpec((XW, BN), lambda i: (0, i)),
        ],
        out_shape=[
            jax.ShapeDtypeStruct((NP, D), jnp.float32),
            jax.ShapeDtypeStruct((XW, NP), jnp.float32),
        ],
    )(h, *part_args, coordT4, Wn1[:D], Wn1[D:],
      bn1.reshape(1, H), Wn2, bn2.reshape(1, D))


def kernel(h, edge_index, coord, edge_mask, We1, be1, We2, be2,
           Wn1, bn1, Wn2, bn2, Wc1, bc1, Wc2):
    N, D = h.shape
    H = We2.shape[0]
    E = edge_index.shape[1]
    row = edge_index[0].astype(jnp.int32)
    col = edge_index[1].astype(jnp.int32)
    coordT = jnp.transpose(coord).astype(jnp.float32).reshape(3 * N)

    P, Qm = _pq(h, We1, be1)
    g = np.empty(H, np.int32)
    for j in range(H // 32):
        for i in range(16):
            g[32 * j + 2 * i] = 32 * j + i
            g[32 * j + 2 * i + 1] = 32 * j + 16 + i
    wr = We1[2 * D].reshape(1, H)[:, g]
    We2g = We2[g, :]
    NSPLIT = 3
    step = (E // NSPLIT // 2560) * 2560
    bounds = [i * step for i in range(NSPLIT)] + [E]
    parts = []
    for lo, hi in zip(bounds[:-1], bounds[1:]):
        rw = row[lo:hi]
        pre, rad, dx, dy, dz = _sc_gather(P, Qm, coordT, rw, col[lo:hi])
        pre = jax.lax.bitcast_convert_type(pre, jnp.bfloat16).reshape(hi - lo, H)
        ef, s = _edge_mlp(pre, rad, edge_mask[lo:hi], wr, We2g, be2, Wc1, bc1, Wc2)
        parts.append(_sc_scatter(ef, s.reshape(hi - lo), dx, dy, dz, rw, N))
    NP = parts[0][0].shape[1]
    parts = [(aggh, aggc.reshape(NC, 8, NP)) for aggh, aggc in parts]
    coordT4 = jnp.zeros((8, NP), jnp.float32).at[:3, :N].set(coordT.reshape(3, N))
    h_pad = jnp.pad(h, ((0, NP - N), (0, 0)))
    h_out, coordoT = _node_mlp(h_pad, parts, coordT4, Wn1, bn1, Wn2, bn2)
    return h_out[:N], jnp.transpose(coordoT[:3, :N])


# speedup vs baseline: 1.9592x; 1.9592x over previous
import functools

import jax
import jax.numpy as jnp
from jax import lax
from jax.experimental import pallas as pl
from jax.experimental.pallas import tpu as pltpu
from jax.experimental.pallas import tpu_sc as plsc

NC = 2
NS = 16
LN = 16
NW = NC * NS

G = 80


def _silu(x):
    return x * jax.nn.sigmoid(x)


def _pq(h, We1, be1):
    N, D = h.shape
    H = We1.shape[1]
    BN = 2000

    def body(h_ref, wa_ref, wb_ref, b_ref, p_ref, q_ref):
        hh = h_ref[...]
        p_ref[...] = jnp.dot(hh, wa_ref[...], preferred_element_type=jnp.float32) + b_ref[...]
        q_ref[...] = jnp.dot(hh, wb_ref[...], preferred_element_type=jnp.float32)

    return pl.pallas_call(
        body,
        grid=(N // BN,),
        in_specs=[
            pl.BlockSpec((BN, D), lambda i: (i, 0)),
            pl.BlockSpec((D, H), lambda i: (0, 0)),
            pl.BlockSpec((D, H), lambda i: (0, 0)),
            pl.BlockSpec((1, H), lambda i: (0, 0)),
        ],
        out_specs=[
            pl.BlockSpec((BN, H), lambda i: (i, 0)),
            pl.BlockSpec((BN, H), lambda i: (i, 0)),
        ],
        out_shape=[
            jax.ShapeDtypeStruct((N, H), jnp.float32),
            jax.ShapeDtypeStruct((N, H), jnp.float32),
        ],
    )(h, We1[:D], We1[D:2 * D], be1.reshape(1, H))


def _sc_gather(P, Q, coordT, row, col):
    N, H = P.shape
    E = row.shape[0]
    EPW = E // NW
    G2 = 40
    nch = EPW // G2
    NPAIR = nch // 2
    mesh = plsc.VectorSubcoreMesh(core_axis_name="c", subcore_axis_name="s")

    vec_e = jax.ShapeDtypeStruct((E,), jnp.float32)
    buf_gh = pltpu.VMEM((G2, H), jnp.float32)
    buf48 = pltpu.VMEM((48,), jnp.float32)

    @functools.partial(
        pl.kernel,
        out_type=[jax.ShapeDtypeStruct((E, H), jnp.float32), vec_e, vec_e, vec_e, vec_e],
        mesh=mesh,
        compiler_params=pltpu.CompilerParams(needs_layout_passes=False),
        scratch_types=[
            pltpu.VMEM((EPW + 16,), jnp.int32),
            pltpu.VMEM((EPW + 16,), jnp.int32),
            pltpu.VMEM((3 * N,), jnp.float32),
            buf_gh, buf_gh, buf_gh, buf_gh, buf_gh, buf_gh,
            buf48, buf48, buf48, buf48, buf48, buf48, buf48, buf48,
            pltpu.SemaphoreType.DMA, pltpu.SemaphoreType.DMA,
            pltpu.SemaphoreType.DMA, pltpu.SemaphoreType.DMA,
        ],
    )
    def k(p_hbm, q_hbm, ct_hbm, row_hbm, col_hbm,
          pre_hbm, rad_hbm, dx_hbm, dy_hbm, dz_hbm,
          idxr, idxc, cv,
          pb0, pb1, qb0, qb1, ob0, ob1,
          rb0, rb1, xb0, xb1, yb0, yb1, zb0, zb1,
          semg0, semg1, semw0, semw1):
        wid = lax.axis_index("c") * NS + lax.axis_index("s")
        base0 = wid * EPW
        n1 = jnp.full((LN,), N, jnp.int32)
        n2 = jnp.full((LN,), 2 * N, jnp.int32)

        pbs, qbs, obs = (pb0, pb1), (qb0, qb1), (ob0, ob1)
        rbs, xbs, ybs, zbs = (rb0, rb1), (xb0, xb1), (yb0, yb1), (zb0, zb1)
        semgs, semws = (semg0, semg1), (semw0, semw1)

        pltpu.sync_copy(row_hbm.at[pl.ds(base0, EPW)], idxr.at[pl.ds(0, EPW)])
        pltpu.sync_copy(col_hbm.at[pl.ds(base0, EPW)], idxc.at[pl.ds(0, EPW)])
        idxr[pl.ds(EPW, LN)] = jnp.zeros((LN,), jnp.int32)
        idxc[pl.ds(EPW, LN)] = jnp.zeros((LN,), jnp.int32)
        pltpu.sync_copy(ct_hbm, cv)

        def issue_gather(kd, b):
            pltpu.async_copy(p_hbm.at[idxr.at[pl.ds(kd * G2, G2)]], pbs[b], semgs[b])
            pltpu.async_copy(q_hbm.at[idxc.at[pl.ds(kd * G2, G2)]], qbs[b], semgs[b])

        def drain_gather(b):
            pltpu.make_async_copy(p_hbm.at[idxr.at[pl.ds(0, G2)]], pbs[b], semgs[b]).wait()
            pltpu.make_async_copy(q_hbm.at[idxc.at[pl.ds(0, G2)]], qbs[b], semgs[b]).wait()

        def drain_write(b):
            pltpu.make_async_copy(obs[b], pre_hbm.at[pl.ds(base0, G2)], semws[b]).wait()
            pltpu.make_async_copy(rbs[b].at[pl.ds(0, G2)], rad_hbm.at[pl.ds(base0, G2)], semws[b]).wait()
            pltpu.make_async_copy(xbs[b].at[pl.ds(0, G2)], dx_hbm.at[pl.ds(base0, G2)], semws[b]).wait()
            pltpu.make_async_copy(ybs[b].at[pl.ds(0, G2)], dy_hbm.at[pl.ds(base0, G2)], semws[b]).wait()
            pltpu.make_async_copy(zbs[b].at[pl.ds(0, G2)], dz_hbm.at[pl.ds(base0, G2)], semws[b]).wait()

        def stage(kd, b, first, issue_next):
            if not first:
                drain_write(b)
            rb, xb, yb, zb = rbs[b], xbs[b], ybs[b], zbs[b]
            base_l = kd * G2
            for g in range(3):
                lsl = pl.ds(base_l + g * LN, LN)
                r16 = idxr[lsl]
                c16 = idxc[lsl]
                dx = plsc.load_gather(cv, [r16]) - plsc.load_gather(cv, [c16])
                dy = plsc.load_gather(cv, [r16 + n1]) - plsc.load_gather(cv, [c16 + n1])
                dz = plsc.load_gather(cv, [r16 + n2]) - plsc.load_gather(cv, [c16 + n2])
                osl = pl.ds(g * LN, LN)
                xb[osl] = dx
                yb[osl] = dy
                zb[osl] = dz
                rb[osl] = dx * dx + dy * dy + dz * dz
            drain_gather(b)
            pb, qb, ob = pbs[b], qbs[b], obs[b]

            def addrow(r, c2):
                for j in range(H // LN):
                    sl2 = pl.ds(j * LN, LN)
                    ob[r, sl2] = pb[r, sl2] + qb[r, sl2]
                return c2

            lax.fori_loop(0, G2, addrow, 0)
            if issue_next:
                issue_gather(kd + 2, b)
            base = base0 + kd * G2
            pltpu.async_copy(ob, pre_hbm.at[pl.ds(base, G2)], semws[b])
            pltpu.async_copy(rb.at[pl.ds(0, G2)], rad_hbm.at[pl.ds(base, G2)], semws[b])
            pltpu.async_copy(xb.at[pl.ds(0, G2)], dx_hbm.at[pl.ds(base, G2)], semws[b])
            pltpu.async_copy(yb.at[pl.ds(0, G2)], dy_hbm.at[pl.ds(base, G2)], semws[b])
            pltpu.async_copy(zb.at[pl.ds(0, G2)], dz_hbm.at[pl.ds(base, G2)], semws[b])

        issue_gather(0, 0)
        issue_gather(1, 1)
        stage(0, 0, True, True)
        stage(1, 1, True, True)

        def pair(kk, c2):
            stage(2 * kk, 0, False, True)
            stage(2 * kk + 1, 1, False, True)
            return c2

        lax.fori_loop(1, NPAIR - 1, pair, 0)
        stage(nch - 2, 0, False, False)
        stage(nch - 1, 1, False, False)
        drain_write(0)
        drain_write(1)

    return k(P, Q, coordT, row, col)


def _edge_mlp(pre, rad, mask, wr, We2, be2, Wc1, bc1, Wc2):
    E, H = pre.shape
    BE = 2560

    def body(pre_ref, rad_ref, m_ref, wr_ref, w2_ref, b2_ref,
             wc1_ref, bc1_ref, wc2_ref, ef_ref, s_ref):
        pre1 = pre_ref[...] + rad_ref[...] * wr_ref[...]
        a1 = _silu(pre1)
        a2 = _silu(jnp.dot(a1, w2_ref[...], preferred_element_type=jnp.float32) + b2_ref[...])
        ef = a2 * m_ref[...]
        c1 = _silu(jnp.dot(ef, wc1_ref[...], preferred_element_type=jnp.float32) + bc1_ref[...])
        sv = jnp.dot(c1, wc2_ref[...], preferred_element_type=jnp.float32) * m_ref[...]
        ef_ref[...] = ef
        s_ref[...] = sv

    full = lambda i: (0, 0)
    return pl.pallas_call(
        body,
        grid=(E // BE,),
        in_specs=[
            pl.BlockSpec((BE, H), lambda i: (i, 0)),
            pl.BlockSpec((BE, 1), lambda i: (i, 0)),
            pl.BlockSpec((BE, 1), lambda i: (i, 0)),
            pl.BlockSpec((1, H), full),
            pl.BlockSpec((H, H), full),
            pl.BlockSpec((1, H), full),
            pl.BlockSpec((H, H), full),
            pl.BlockSpec((1, H), full),
            pl.BlockSpec((H, 1), full),
        ],
        out_specs=[
            pl.BlockSpec((BE, H), lambda i: (i, 0)),
            pl.BlockSpec((BE, 1), lambda i: (i, 0)),
        ],
        out_shape=[
            jax.ShapeDtypeStruct((E, H), jnp.float32),
            jax.ShapeDtypeStruct((E, 1), jnp.float32),
        ],
    )(pre, rad.reshape(E, 1), mask, wr, We2, be2.reshape(1, H), Wc1, bc1.reshape(1, H), Wc2)


def _sc_scatter(ef, s, dx, dy, dz, row, N):
    E, H = ef.shape
    EPW = E // NW
    G2 = 40
    nch = EPW // G2
    NP = ((N + 40 * NS - 1) // (40 * NS)) * (40 * NS)
    RPT = NP // NS
    ZR = 40
    XW = 8
    ZC = XW * RPT // 4
    mesh = plsc.VectorSubcoreMesh(core_axis_name="c", subcore_axis_name="s")

    ibuf40 = pltpu.VMEM((G2,), jnp.int32)
    ibuf48 = pltpu.VMEM((48,), jnp.int32)
    fbuf48 = pltpu.VMEM((48,), jnp.float32)

    @functools.partial(
        pl.kernel,
        out_type=[
            jax.ShapeDtypeStruct((NC, NP, H), jnp.float32),
            jax.ShapeDtypeStruct((NC, XW * NP), jnp.float32),
        ],
        mesh=mesh,
        compiler_params=pltpu.CompilerParams(needs_layout_passes=False),
        scratch_types=[
            ibuf48, ibuf48,
            fbuf48, fbuf48, fbuf48, fbuf48, fbuf48, fbuf48, fbuf48, fbuf48,
            pltpu.VMEM((G2, H), jnp.float32),
            pltpu.VMEM((G2, H), jnp.float32),
            ibuf40, ibuf40, ibuf40, ibuf40, ibuf40, ibuf40, ibuf40, ibuf40,
            fbuf48, fbuf48, fbuf48, fbuf48, fbuf48, fbuf48, fbuf48,
            pltpu.VMEM((ZR, H), jnp.float32),
            pltpu.VMEM((ZC,), jnp.float32),
            pltpu.VMEM_SHARED((NP, H), jnp.float32),
            pltpu.VMEM_SHARED((XW * NP,), jnp.float32),
            pltpu.SemaphoreType.DMA, pltpu.SemaphoreType.DMA,
            pltpu.SemaphoreType.DMA, pltpu.SemaphoreType.DMA,
        ],
    )
    def k(ef_hbm, s_hbm, dx_hbm, dy_hbm, dz_hbm, row_hbm,
          aggh_hbm, aggc_hbm,
          rw0, rw1, sb0, sb1, xb0, xb1, yb0, yb1, zx0, zx1,
          ef0, ef1,
          ix0, ix1, iy0, iy1, iz0, iz1, ic0, ic1,
          px0, px1, py0, py1, pz0, pz1, pone,
          zb, zcb, acc_h, acc_c,
          semr0, semr1, sems0, sems1):
        cid = lax.axis_index("c")
        sid = lax.axis_index("s")
        wid = cid * NS + sid
        base0 = wid * EPW
        fz16 = jnp.zeros((LN,), jnp.float32)
        ones16 = jnp.ones((LN,), jnp.float32)
        np1 = jnp.full((LN,), NP, jnp.int32)
        np2 = jnp.full((LN,), 2 * NP, jnp.int32)
        np3 = jnp.full((LN,), 3 * NP, jnp.int32)
        rws, sbs = (rw0, rw1), (sb0, sb1)
        xbs, ybs, zxs = (xb0, xb1), (yb0, yb1), (zx0, zx1)
        efs = (ef0, ef1)
        ixs, iys, izs, ics = (ix0, ix1), (iy0, iy1), (iz0, iz1), (ic0, ic1)
        pxs, pys, pzs = (px0, px1), (py0, py1), (pz0, pz1)
        semrs, semss = (semr0, semr1), (sems0, sems1)

        def zrow(r, c2):
            for j in range(H // LN):
                zb[r, pl.ds(j * LN, LN)] = fz16
            return c2

        lax.fori_loop(0, ZR, zrow, 0)

        def zflat(i, c2):
            zcb[pl.ds(i * LN, LN)] = fz16
            return c2

        lax.fori_loop(0, ZC // LN, zflat, 0)
        for g in range(3):
            pone[pl.ds(g * LN, LN)] = ones16

        r0 = sid * RPT
        for t in range(RPT // ZR):
            pltpu.sync_copy(zb, acc_h.at[pl.ds(r0 + t * ZR, ZR)])
        for t in range(4):
            pltpu.sync_copy(zcb, acc_c.at[pl.ds(sid * XW * RPT + t * ZC, ZC)])
        plsc.subcore_barrier()

        iota16 = lax.iota(jnp.int32, LN)
        tailm = iota16 < (G2 - 2 * LN)
        tidx = iota16 + 2 * LN

        def issue_reads(kd, b):
            base = base0 + kd * G2
            pltpu.async_copy(ef_hbm.at[pl.ds(base, G2)], efs[b], semrs[b])
            pltpu.async_copy(row_hbm.at[pl.ds(base, G2)], rws[b].at[pl.ds(0, G2)], semrs[b])
            pltpu.async_copy(s_hbm.at[pl.ds(base, G2)], sbs[b].at[pl.ds(0, G2)], semrs[b])
            pltpu.async_copy(dx_hbm.at[pl.ds(base, G2)], xbs[b].at[pl.ds(0, G2)], semrs[b])
            pltpu.async_copy(dy_hbm.at[pl.ds(base, G2)], ybs[b].at[pl.ds(0, G2)], semrs[b])
            pltpu.async_copy(dz_hbm.at[pl.ds(base, G2)], zxs[b].at[pl.ds(0, G2)], semrs[b])

        def drain_reads(b):
            pltpu.make_async_copy(ef_hbm.at[pl.ds(base0, G2)], efs[b], semrs[b]).wait()
            pltpu.make_async_copy(row_hbm.at[pl.ds(base0, G2)], rws[b].at[pl.ds(0, G2)], semrs[b]).wait()
            pltpu.make_async_copy(s_hbm.at[pl.ds(base0, G2)], sbs[b].at[pl.ds(0, G2)], semrs[b]).wait()
            pltpu.make_async_copy(dx_hbm.at[pl.ds(base0, G2)], xbs[b].at[pl.ds(0, G2)], semrs[b]).wait()
            pltpu.make_async_copy(dy_hbm.at[pl.ds(base0, G2)], ybs[b].at[pl.ds(0, G2)], semrs[b]).wait()
            pltpu.make_async_copy(dz_hbm.at[pl.ds(base0, G2)], zxs[b].at[pl.ds(0, G2)], semrs[b]).wait()

        def drain_adds(b):
            pltpu.make_async_copy(efs[b], acc_h.at[ixs[b]], semss[b]).wait()
            pltpu.make_async_copy(pxs[b].at[pl.ds(0, G2)], acc_c.at[ixs[b]], semss[b]).wait()
            pltpu.make_async_copy(pys[b].at[pl.ds(0, G2)], acc_c.at[iys[b]], semss[b]).wait()
            pltpu.make_async_copy(pzs[b].at[pl.ds(0, G2)], acc_c.at[izs[b]], semss[b]).wait()
            pltpu.make_async_copy(pone.at[pl.ds(0, G2)], acc_c.at[ics[b]], semss[b]).wait()

        def stage(kd, b, first, issue_next):
            if not first:
                drain_adds(1 - b)
            if issue_next:
                issue_reads(kd + 1, 1 - b)
            ixv, iyv, izv, icv = ixs[b], iys[b], izs[b], ics[b]
            pxv, pyv, pzv = pxs[b], pys[b], pzs[b]
            drain_reads(b)
            for g in range(3):
                lsl = pl.ds(g * LN, LN)
                v16 = rws[b][lsl]
                sv = sbs[b][lsl]
                pxe = xbs[b][lsl] * sv
                pye = ybs[b][lsl] * sv
                pze = zxs[b][lsl] * sv
                if g < 2:
                    ixv[lsl] = v16
                    iyv[lsl] = v16 + np1
                    izv[lsl] = v16 + np2
                    icv[lsl] = v16 + np3
                    pxv[lsl] = pxe
                    pyv[lsl] = pye
                    pzv[lsl] = pze
                else:
                    plsc.store_scatter(ixv, [tidx], v16, mask=tailm)
                    plsc.store_scatter(iyv, [tidx], v16 + np1, mask=tailm)
                    plsc.store_scatter(izv, [tidx], v16 + np2, mask=tailm)
                    plsc.store_scatter(icv, [tidx], v16 + np3, mask=tailm)
                    pxv[lsl] = pxe
                    pyv[lsl] = pye
                    pzv[lsl] = pze
            pltpu.async_copy(efs[b], acc_h.at[ixv], semss[b], add=True)
            pltpu.async_copy(pxv.at[pl.ds(0, G2)], acc_c.at[ixv], semss[b], add=True)
            pltpu.async_copy(pyv.at[pl.ds(0, G2)], acc_c.at[iyv], semss[b], add=True)
            pltpu.async_copy(pzv.at[pl.ds(0, G2)], acc_c.at[izv], semss[b], add=True)
            pltpu.async_copy(pone.at[pl.ds(0, G2)], acc_c.at[icv], semss[b], add=True)

        issue_reads(0, 0)
        stage(0, 0, True, True)

        def pair(kk, c2):
            stage(2 * kk + 1, 1, False, True)
            stage(2 * kk + 2, 0, False, True)
            return c2

        lax.fori_loop(0, (nch - 2) // 2, pair, 0)
        stage(nch - 1, 1, False, False)
        drain_adds(1)
        plsc.subcore_barrier()
        pltpu.sync_copy(acc_h.at[pl.ds(r0, RPT)], aggh_hbm.at[cid, pl.ds(r0, RPT)])
        pltpu.sync_copy(acc_c.at[pl.ds(sid * XW * RPT, XW * RPT)],
                        aggc_hbm.at[cid, pl.ds(sid * XW * RPT, XW * RPT)])

    return k(ef, s, dx, dy, dz, row)


def _node_mlp(h, parts, coordT4, Wn1, bn1, Wn2, bn2):
    NP, D = h.shape
    H = Wn2.shape[0]
    XW = parts[0][1].shape[1]
    NPART = len(parts)
    BN = 2048

    def body(*refs):
        h_ref = refs[0]
        part_refs = refs[1:1 + 2 * NPART]
        cp_ref, wa_ref, wb_ref, b1_ref, w2_ref, b2_ref, ho_ref, co_ref = refs[1 + 2 * NPART:]
        agg = 0.0
        c4 = 0.0
        for i in range(NPART):
            agg = agg + part_refs[2 * i][0] + part_refs[2 * i][1]
            c4 = c4 + part_refs[2 * i + 1][0] + part_refs[2 * i + 1][1]
        x = _silu(jnp.dot(h_ref[...], wa_ref[...], preferred_element_type=jnp.float32)
                  + jnp.dot(agg, wb_ref[...], preferred_element_type=jnp.float32)
                  + b1_ref[...])
        ho_ref[...] = jnp.dot(x, w2_ref[...], preferred_element_type=jnp.float32) + b2_ref[...]
        cnt = jnp.maximum(c4[3:4, :], 1.0)
        co_ref[...] = cp_ref[...] + c4 / cnt

    full = lambda i: (0, 0)
    part_specs = []
    part_args = []
    for aggh, aggc in parts:
        part_specs.append(pl.BlockSpec((2, BN, H), lambda i: (0, i, 0)))
        part_specs.append(pl.BlockSpec((2, XW, BN), lambda i: (0, 0, i)))
        part_args.extend([aggh, aggc])
    return pl.pallas_call(
        body,
        grid=(NP // BN,),
        in_specs=[pl.BlockSpec((BN, D), lambda i: (i, 0))] + part_specs + [
            pl.BlockSpec((XW, BN), lambda i: (0, i)),
            pl.BlockSpec((D, H), full),
            pl.BlockSpec((H, H), full),
            pl.BlockSpec((1, H), full),
            pl.BlockSpec((H, D), full),
            pl.BlockSpec((1, D), full),
        ],
        out_specs=[
            pl.BlockSpec((BN, D), lambda i: (i, 0)),
            pl.BlockSpec((XW, BN), lambda i: (0, i)),
        ],
        out_shape=[
            jax.ShapeDtypeStruct((NP, D), jnp.float32),
            jax.ShapeDtypeStruct((XW, NP), jnp.float32),
        ],
    )(h, *part_args, coordT4, Wn1[:D], Wn1[D:],
      bn1.reshape(1, H), Wn2, bn2.reshape(1, D))


def kernel(h, edge_index, coord, edge_mask, We1, be1, We2, be2,
           Wn1, bn1, Wn2, bn2, Wc1, bc1, Wc2):
    N, D = h.shape
    H = We2.shape[0]
    E = edge_index.shape[1]
    row = edge_index[0].astype(jnp.int32)
    col = edge_index[1].astype(jnp.int32)
    coordT = jnp.transpose(coord).astype(jnp.float32).reshape(3 * N)

    P, Qm = _pq(h, We1, be1)
    wr = We1[2 * D].reshape(1, H)
    NSPLIT = 3
    step = (E // NSPLIT // 2560) * 2560
    bounds = [i * step for i in range(NSPLIT)] + [E]
    parts = []
    for lo, hi in zip(bounds[:-1], bounds[1:]):
        rw = row[lo:hi]
        pre, rad, dx, dy, dz = _sc_gather(P, Qm, coordT, rw, col[lo:hi])
        ef, s = _edge_mlp(pre, rad, edge_mask[lo:hi], wr, We2, be2, Wc1, bc1, Wc2)
        parts.append(_sc_scatter(ef, s.reshape(hi - lo), dx, dy, dz, rw, N))
    NP = parts[0][0].shape[1]
    parts = [(aggh, aggc.reshape(NC, 8, NP)) for aggh, aggc in parts]
    coordT4 = jnp.zeros((8, NP), jnp.float32).at[:3, :N].set(coordT.reshape(3, N))
    h_pad = jnp.pad(h, ((0, NP - N), (0, 0)))
    h_out, coordoT = _node_mlp(h_pad, parts, coordT4, Wn1, bn1, Wn2, bn2)
    return h_out[:N], jnp.transpose(coordoT[:3, :N])
